# baseline (speedup 1.0000x reference)
import functools

import jax
import jax.numpy as jnp
from jax import lax
from jax.experimental import pallas as pl
from jax.experimental.pallas import tpu as pltpu
from jax.experimental.pallas import tpu_sc as plsc

DIM = 1024
HEADS = 4
NUM_MEM = 65536
NUM_KEYS = 256
DQK = 512
DV = 512
TOPK = 8
DH_QK = DQK // HEADS
DH_V = DV // HEADS
RMS_EPS = 1.1920929e-07
IDW_EPS = 0.001

N_TOK = 2 * 2048
TB = 512
GRID = N_TOK // TB
KH = HEADS * TOPK
N_ROWS = N_TOK * KH

BIG = 3.0e38


def _stage1_body(x_ref, wq_ref, gq_ref, k1_ref, k2_ref, wg_ref, wv_ref,
                 gg_ref, gv_ref, idx_ref, w_ref, tvs_ref, g_ref):
    x = x_ref[:]
    inv = lax.rsqrt(jnp.mean(x * x, axis=-1, keepdims=True) + RMS_EPS)
    xni = x * inv
    xn = xni * gq_ref[:]
    q = jnp.dot(xn, wq_ref[:], preferred_element_type=jnp.float32)

    zg = jnp.dot(xni * gg_ref[:], wg_ref[:],
                 preferred_element_type=jnp.float32)
    g_ref[:] = 1.0 / (1.0 + jnp.exp(-zg))
    tv = jnp.dot(xni * gv_ref[:], wv_ref[:],
                 preferred_element_type=jnp.float32)
    parts = []
    for h in range(HEADS):
        tvh = tv[:, h * DH_V:(h + 1) * DH_V]
        mu = jnp.mean(tvh, axis=-1, keepdims=True)
        c = tvh - mu
        std = jnp.sqrt(jnp.sum(c * c, axis=-1, keepdims=True) / (DH_V - 1))
        parts.append(c / jnp.maximum(std, 1e-10))
    tvs_ref[:] = jnp.concatenate(parts, axis=1)

    dists = []
    for which, k_ref in ((0, k1_ref), (1, k2_ref)):
        for h in range(HEADS):
            qh = q[:, which * DQK + h * DH_QK: which * DQK + (h + 1) * DH_QK]
            kh = k_ref[h]
            cross = lax.dot_general(
                qh, kh, (((1,), (1,)), ((), ())),
                preferred_element_type=jnp.float32)
            qs = jnp.sum(qh * qh, axis=-1, keepdims=True)
            ks = jnp.sum(kh * kh, axis=-1)[None, :]
            dists.append(lax.transpose(qs + ks - 2.0 * cross, (1, 0)))
    cur = jnp.concatenate(dists, axis=1)

    iota_k = lax.broadcasted_iota(jnp.int32, (NUM_KEYS, 8 * TB), 0)
    vals, sels = [], []
    for t in range(TOPK):
        m = jnp.min(cur, axis=0, keepdims=True)
        eq = cur == m
        sel = jnp.min(jnp.where(eq, iota_k, NUM_KEYS + 1), axis=0,
                      keepdims=True)
        vals.append(m)
        sels.append(sel)
        if t + 1 < TOPK:
            cur = jnp.where(iota_k == sel, BIG, cur)
    s_all = jnp.concatenate(vals, axis=0) + IDW_EPS
    i_all = jnp.concatenate(sels, axis=0)
    ls_all = -jnp.log(s_all)

    HB = HEADS * TB
    s1, s2 = s_all[:, :HB], s_all[:, HB:]
    l1, l2 = ls_all[:, :HB], ls_all[:, HB:]
    i1, i2 = i_all[:, :HB], i_all[:, HB:]
    rep = lambda a: jnp.concatenate(
        [jnp.broadcast_to(a[i:i + 1], (TOPK, HB)) for i in range(TOPK)],
        axis=0)
    til = lambda a: jnp.concatenate([a] * TOPK, axis=0)
    curp = rep(s1) * til(s2)
    sc64 = rep(l1) + til(l2)
    ix64 = rep(i1) * NUM_KEYS + til(i2)

    iota_c = lax.broadcasted_iota(jnp.int32, (TOPK * TOPK, HB), 0)
    fis, fss = [], []
    for t in range(TOPK):
        m = jnp.min(curp, axis=0, keepdims=True)
        eq = curp == m
        sel = jnp.min(jnp.where(eq, iota_c, TOPK * TOPK + 1), axis=0,
                      keepdims=True)
        oh = iota_c == sel
        fis.append(jnp.sum(jnp.where(oh, ix64, 0), axis=0, keepdims=True))
        fss.append(jnp.sum(jnp.where(oh, sc64, 0.0), axis=0, keepdims=True))
        if t + 1 < TOPK:
            curp = jnp.where(oh, BIG, curp)
    fi = jnp.concatenate(fis, axis=0)
    fs = jnp.concatenate(fss, axis=0)
    m = jnp.max(fs, axis=0, keepdims=True)
    e = jnp.exp(fs - m)
    w8 = e / jnp.sum(e, axis=0, keepdims=True)

    idx_ref[:] = jnp.concatenate(
        [fi[:, h * TB:(h + 1) * TB] * HEADS + h for h in range(HEADS)],
        axis=0)
    w_ref[:] = jnp.concatenate(
        [w8[:, h * TB:(h + 1) * TB] for h in range(HEADS)], axis=0)


def _stage1(x, Wq, gq, k1, k2, Wg, Wv, gg, gv):
    return pl.pallas_call(
        _stage1_body,
        grid=(GRID,),
        in_specs=[
            pl.BlockSpec((TB, DIM), lambda i: (i, 0)),
            pl.BlockSpec((DIM, 2 * DQK), lambda i: (0, 0)),
            pl.BlockSpec((1, DIM), lambda i: (0, 0)),
            pl.BlockSpec((HEADS, NUM_KEYS, DH_QK), lambda i: (0, 0, 0)),
            pl.BlockSpec((HEADS, NUM_KEYS, DH_QK), lambda i: (0, 0, 0)),
            pl.BlockSpec((DIM, 1), lambda i: (0, 0)),
            pl.BlockSpec((DIM, DV), lambda i: (0, 0)),
            pl.BlockSpec((1, DIM), lambda i: (0, 0)),
            pl.BlockSpec((1, DIM), lambda i: (0, 0)),
        ],
        out_specs=[
            pl.BlockSpec((KH, TB), lambda i: (0, i)),
            pl.BlockSpec((KH, TB), lambda i: (0, i)),
            pl.BlockSpec((TB, DV), lambda i: (i, 0)),
            pl.BlockSpec((TB, 1), lambda i: (i, 0)),
        ],
        out_shape=[
            jax.ShapeDtypeStruct((KH, N_TOK), jnp.int32),
            jax.ShapeDtypeStruct((KH, N_TOK), jnp.float32),
            jax.ShapeDtypeStruct((N_TOK, DV), jnp.float32),
            jax.ShapeDtypeStruct((N_TOK, 1), jnp.float32),
        ],
    )(x, Wq, gq.reshape(1, DIM), k1, k2, Wg, Wv,
      gg.reshape(1, DIM), gv.reshape(1, DIM))


_NC, _NS = 2, 16
_NW = _NC * _NS
_RPW = N_ROWS // _NW
_CH = 128
_NCH = _RPW // _CH


_NBUF = 3


def _sc_gather_body(table_hbm, idx_hbm, out_hbm, idx_v, rows_v,
                    gs0, gs1, gs2, ss0, ss1, ss2):
    wid = lax.axis_index("s") * _NC + lax.axis_index("c")
    base = wid * _RPW
    pltpu.sync_copy(idx_hbm.at[wid], idx_v)
    gsems = (gs0, gs1, gs2)
    ssems = (ss0, ss1, ss2)
    gcp = [None] * _NCH
    scp = [None] * _NCH
    for j in range(_NBUF):
        gcp[j] = pltpu.async_copy(
            table_hbm.at[idx_v.at[j]], rows_v.at[j], gsems[j])
    for j in range(_NCH):
        b = j % _NBUF
        gcp[j].wait()
        scp[j] = pltpu.async_copy(
            rows_v.at[b], out_hbm.at[pl.ds(base + j * _CH, _CH)], ssems[b])
        if j + _NBUF < _NCH:
            scp[j].wait()
            gcp[j + _NBUF] = pltpu.async_copy(
                table_hbm.at[idx_v.at[j + _NBUF]], rows_v.at[b], gsems[b])
    for j in range(_NCH - _NBUF, _NCH):
        scp[j].wait()


@functools.lru_cache(maxsize=1)
def _sc_gather_kernel():
    return pl.kernel(
        _sc_gather_body,
        mesh=plsc.VectorSubcoreMesh(core_axis_name="c", subcore_axis_name="s"),
        out_type=jax.ShapeDtypeStruct((N_ROWS, DH_V), jnp.float32),
        scratch_types=[
            pltpu.VMEM((_NCH, _CH), jnp.int32),
            pltpu.VMEM((_NBUF, _CH, DH_V), jnp.float32),
            pltpu.SemaphoreType.DMA,
            pltpu.SemaphoreType.DMA,
            pltpu.SemaphoreType.DMA,
            pltpu.SemaphoreType.DMA,
            pltpu.SemaphoreType.DMA,
            pltpu.SemaphoreType.DMA,
        ],
    )


def _stage3b_body(r_ref, w_ref, tvs_ref, g_ref, wo_ref, go_ref, out_ref):
    wt = lax.transpose(w_ref[:], (1, 0))
    g = g_ref[:]
    tvs = tvs_ref[:]
    parts = []
    for h in range(HEADS):
        vh = jnp.zeros((TB, DH_V), jnp.float32)
        for k in range(TOPK):
            j = h * TOPK + k
            vh = vh + wt[:, j:j + 1] * r_ref[j]
        tvsh = tvs[:, h * DH_V:(h + 1) * DH_V]
        parts.append(tvsh + g * (vh - tvsh))
    o = jnp.concatenate(parts, axis=1)
    inv2 = lax.rsqrt(jnp.mean(o * o, axis=-1, keepdims=True) + RMS_EPS)
    y = o * inv2 * go_ref[:]
    out_ref[:] = jnp.dot(y, wo_ref[:], preferred_element_type=jnp.float32)


def _stage3b(rows3, w, tvs, g, Wo, go):
    return pl.pallas_call(
        _stage3b_body,
        grid=(GRID,),
        in_specs=[
            pl.BlockSpec((KH, TB, DH_V), lambda i: (0, i, 0)),
            pl.BlockSpec((KH, TB), lambda i: (0, i)),
            pl.BlockSpec((TB, DV), lambda i: (i, 0)),
            pl.BlockSpec((TB, 1), lambda i: (i, 0)),
            pl.BlockSpec((DV, DIM), lambda i: (0, 0)),
            pl.BlockSpec((1, DV), lambda i: (0, 0)),
        ],
        out_specs=pl.BlockSpec((TB, DIM), lambda i: (i, 0)),
        out_shape=jax.ShapeDtypeStruct((N_TOK, DIM), jnp.float32),
    )(rows3, w, tvs, g, Wo, go.reshape(1, DV))


def kernel(tokens, memories, keys, Wq, Wg, Wv, Wo, gq, gg, gv, go):
    b, n, _ = tokens.shape
    x = tokens.reshape(b * n, DIM)
    idx, w, tvs, g = _stage1(x, Wq, gq, keys[0], keys[1], Wg, Wv, gg, gv)
    table = memories.reshape(NUM_MEM * HEADS, DH_V)
    rows = _sc_gather_kernel()(table, idx.reshape(_NW, _NCH, _CH))
    out = _stage3b(rows.reshape(KH, N_TOK, DH_V), w, tvs, g, Wo, go)
    return out.reshape(b, n, DIM)


# speedup vs baseline: 1.0316x; 1.0316x over previous
import functools

import jax
import jax.numpy as jnp
from jax import lax
from jax.experimental import pallas as pl
from jax.experimental.pallas import tpu as pltpu
from jax.experimental.pallas import tpu_sc as plsc

DIM = 1024
HEADS = 4
NUM_MEM = 65536
NUM_KEYS = 256
DQK = 512
DV = 512
TOPK = 8
DH_QK = DQK // HEADS
DH_V = DV // HEADS
RMS_EPS = 1.1920929e-07
IDW_EPS = 0.001

N_TOK = 2 * 2048
TB = 512
GRID = N_TOK // TB
KH = HEADS * TOPK
N_ROWS = N_TOK * KH

BIG = 3.0e38


def _stage1_body(x_ref, wq_ref, gq_ref, k1_ref, k2_ref, idx_ref, w_ref):
    x = x_ref[:]
    inv = lax.rsqrt(jnp.mean(x * x, axis=-1, keepdims=True) + RMS_EPS)
    xn = x * inv * gq_ref[:]
    q = jnp.dot(xn, wq_ref[:], preferred_element_type=jnp.float32)

    dists = []
    for which, k_ref in ((0, k1_ref), (1, k2_ref)):
        for h in range(HEADS):
            qh = q[:, which * DQK + h * DH_QK: which * DQK + (h + 1) * DH_QK]
            kh = k_ref[h]
            cross = lax.dot_general(
                qh, kh, (((1,), (1,)), ((), ())),
                preferred_element_type=jnp.float32)
            qs = jnp.sum(qh * qh, axis=-1, keepdims=True)
            ks = jnp.sum(kh * kh, axis=-1)[None, :]
            dists.append(lax.transpose(qs + ks - 2.0 * cross, (1, 0)))
    cur = jnp.concatenate(dists, axis=1)

    iota_k = lax.broadcasted_iota(jnp.int32, (NUM_KEYS, 8 * TB), 0)
    vals, sels = [], []
    for t in range(TOPK):
        m = jnp.min(cur, axis=0, keepdims=True)
        eq = cur == m
        sel = jnp.min(jnp.where(eq, iota_k, NUM_KEYS + 1), axis=0,
                      keepdims=True)
        vals.append(m)
        sels.append(sel)
        if t + 1 < TOPK:
            cur = jnp.where(iota_k == sel, BIG, cur)
    s_all = jnp.concatenate(vals, axis=0) + IDW_EPS
    i_all = jnp.concatenate(sels, axis=0)
    ls_all = -jnp.log(s_all)

    HB = HEADS * TB
    s1, s2 = s_all[:, :HB], s_all[:, HB:]
    l1, l2 = ls_all[:, :HB], ls_all[:, HB:]
    i1, i2 = i_all[:, :HB], i_all[:, HB:]
    rep = lambda a: jnp.concatenate(
        [jnp.broadcast_to(a[i:i + 1], (TOPK, HB)) for i in range(TOPK)],
        axis=0)
    til = lambda a: jnp.concatenate([a] * TOPK, axis=0)
    curp = rep(s1) * til(s2)
    sc64 = rep(l1) + til(l2)
    ix64 = rep(i1) * NUM_KEYS + til(i2)

    iota_c = lax.broadcasted_iota(jnp.int32, (TOPK * TOPK, HB), 0)
    fis, fss = [], []
    for t in range(TOPK):
        m = jnp.min(curp, axis=0, keepdims=True)
        eq = curp == m
        sel = jnp.min(jnp.where(eq, iota_c, TOPK * TOPK + 1), axis=0,
                      keepdims=True)
        oh = iota_c == sel
        fis.append(jnp.sum(jnp.where(oh, ix64, 0), axis=0, keepdims=True))
        fss.append(jnp.sum(jnp.where(oh, sc64, 0.0), axis=0, keepdims=True))
        if t + 1 < TOPK:
            curp = jnp.where(oh, BIG, curp)
    fi = jnp.concatenate(fis, axis=0)
    fs = jnp.concatenate(fss, axis=0)
    m = jnp.max(fs, axis=0, keepdims=True)
    e = jnp.exp(fs - m)
    w8 = e / jnp.sum(e, axis=0, keepdims=True)

    idx_ref[:] = jnp.concatenate(
        [fi[:, h * TB:(h + 1) * TB] * HEADS + h for h in range(HEADS)],
        axis=0)
    w_ref[:] = jnp.concatenate(
        [w8[:, h * TB:(h + 1) * TB] for h in range(HEADS)], axis=0)


def _stage1(x, Wq, gq, k1, k2):
    return pl.pallas_call(
        _stage1_body,
        grid=(GRID,),
        in_specs=[
            pl.BlockSpec((TB, DIM), lambda i: (i, 0)),
            pl.BlockSpec((DIM, 2 * DQK), lambda i: (0, 0)),
            pl.BlockSpec((1, DIM), lambda i: (0, 0)),
            pl.BlockSpec((HEADS, NUM_KEYS, DH_QK), lambda i: (0, 0, 0)),
            pl.BlockSpec((HEADS, NUM_KEYS, DH_QK), lambda i: (0, 0, 0)),
        ],
        out_specs=[
            pl.BlockSpec((KH, TB), lambda i: (0, i)),
            pl.BlockSpec((KH, TB), lambda i: (0, i)),
        ],
        out_shape=[
            jax.ShapeDtypeStruct((KH, N_TOK), jnp.int32),
            jax.ShapeDtypeStruct((KH, N_TOK), jnp.float32),
        ],
    )(x, Wq, gq.reshape(1, DIM), k1, k2)


def _stage3a_body(x_ref, wg_ref, wv_ref, gg_ref, gv_ref, tvs_ref, g_ref):
    x = x_ref[:]
    inv = lax.rsqrt(jnp.mean(x * x, axis=-1, keepdims=True) + RMS_EPS)
    xn = x * inv
    zg = jnp.dot(xn * gg_ref[:], wg_ref[:],
                 preferred_element_type=jnp.float32)
    g_ref[:] = 1.0 / (1.0 + jnp.exp(-zg))
    tv = jnp.dot(xn * gv_ref[:], wv_ref[:],
                 preferred_element_type=jnp.float32)
    parts = []
    for h in range(HEADS):
        tvh = tv[:, h * DH_V:(h + 1) * DH_V]
        mu = jnp.mean(tvh, axis=-1, keepdims=True)
        c = tvh - mu
        std = jnp.sqrt(jnp.sum(c * c, axis=-1, keepdims=True) / (DH_V - 1))
        parts.append(c / jnp.maximum(std, 1e-10))
    tvs_ref[:] = jnp.concatenate(parts, axis=1)


def _stage3a(x, Wg, Wv, gg, gv):
    return pl.pallas_call(
        _stage3a_body,
        grid=(GRID,),
        in_specs=[
            pl.BlockSpec((TB, DIM), lambda i: (i, 0)),
            pl.BlockSpec((DIM, 1), lambda i: (0, 0)),
            pl.BlockSpec((DIM, DV), lambda i: (0, 0)),
            pl.BlockSpec((1, DIM), lambda i: (0, 0)),
            pl.BlockSpec((1, DIM), lambda i: (0, 0)),
        ],
        out_specs=[
            pl.BlockSpec((TB, DV), lambda i: (i, 0)),
            pl.BlockSpec((TB, 1), lambda i: (i, 0)),
        ],
        out_shape=[
            jax.ShapeDtypeStruct((N_TOK, DV), jnp.float32),
            jax.ShapeDtypeStruct((N_TOK, 1), jnp.float32),
        ],
    )(x, Wg, Wv, gg.reshape(1, DIM), gv.reshape(1, DIM))


_NC, _NS = 2, 16
_NW = _NC * _NS
_RPW = N_ROWS // _NW
_CH = 128
_NCH = _RPW // _CH


_NBUF = 3


def _sc_gather_body(table_hbm, idx_hbm, out_hbm, idx_v, rows_v,
                    gs0, gs1, gs2, ss0, ss1, ss2):
    wid = lax.axis_index("s") * _NC + lax.axis_index("c")
    base = wid * _RPW
    pltpu.sync_copy(idx_hbm.at[wid], idx_v)
    gsems = (gs0, gs1, gs2)
    ssems = (ss0, ss1, ss2)
    gcp = [None] * _NCH
    scp = [None] * _NCH
    for j in range(_NBUF):
        gcp[j] = pltpu.async_copy(
            table_hbm.at[idx_v.at[j]], rows_v.at[j], gsems[j])
    for j in range(_NCH):
        b = j % _NBUF
        gcp[j].wait()
        scp[j] = pltpu.async_copy(
            rows_v.at[b], out_hbm.at[pl.ds(base + j * _CH, _CH)], ssems[b])
        if j + _NBUF < _NCH:
            scp[j].wait()
            gcp[j + _NBUF] = pltpu.async_copy(
                table_hbm.at[idx_v.at[j + _NBUF]], rows_v.at[b], gsems[b])
    for j in range(_NCH - _NBUF, _NCH):
        scp[j].wait()


@functools.lru_cache(maxsize=1)
def _sc_gather_kernel():
    return pl.kernel(
        _sc_gather_body,
        mesh=plsc.VectorSubcoreMesh(core_axis_name="c", subcore_axis_name="s"),
        out_type=jax.ShapeDtypeStruct((N_ROWS, DH_V), jnp.float32),
        scratch_types=[
            pltpu.VMEM((_NCH, _CH), jnp.int32),
            pltpu.VMEM((_NBUF, _CH, DH_V), jnp.float32),
            pltpu.SemaphoreType.DMA,
            pltpu.SemaphoreType.DMA,
            pltpu.SemaphoreType.DMA,
            pltpu.SemaphoreType.DMA,
            pltpu.SemaphoreType.DMA,
            pltpu.SemaphoreType.DMA,
        ],
    )


def _stage3b_body(r_ref, w_ref, tvs_ref, g_ref, wo_ref, go_ref, out_ref):
    wt = lax.transpose(w_ref[:], (1, 0))
    g = g_ref[:]
    tvs = tvs_ref[:]
    parts = []
    for h in range(HEADS):
        vh = jnp.zeros((TB, DH_V), jnp.float32)
        for k in range(TOPK):
            j = h * TOPK + k
            vh = vh + wt[:, j:j + 1] * r_ref[j]
        tvsh = tvs[:, h * DH_V:(h + 1) * DH_V]
        parts.append(tvsh + g * (vh - tvsh))
    o = jnp.concatenate(parts, axis=1)
    inv2 = lax.rsqrt(jnp.mean(o * o, axis=-1, keepdims=True) + RMS_EPS)
    y = o * inv2 * go_ref[:]
    out_ref[:] = jnp.dot(y, wo_ref[:], preferred_element_type=jnp.float32)


def _stage3b(rows3, w, tvs, g, Wo, go):
    return pl.pallas_call(
        _stage3b_body,
        grid=(GRID,),
        in_specs=[
            pl.BlockSpec((KH, TB, DH_V), lambda i: (0, i, 0)),
            pl.BlockSpec((KH, TB), lambda i: (0, i)),
            pl.BlockSpec((TB, DV), lambda i: (i, 0)),
            pl.BlockSpec((TB, 1), lambda i: (i, 0)),
            pl.BlockSpec((DV, DIM), lambda i: (0, 0)),
            pl.BlockSpec((1, DV), lambda i: (0, 0)),
        ],
        out_specs=pl.BlockSpec((TB, DIM), lambda i: (i, 0)),
        out_shape=jax.ShapeDtypeStruct((N_TOK, DIM), jnp.float32),
    )(rows3, w, tvs, g, Wo, go.reshape(1, DV))


def kernel(tokens, memories, keys, Wq, Wg, Wv, Wo, gq, gg, gv, go):
    b, n, _ = tokens.shape
    x = tokens.reshape(b * n, DIM)
    idx, w = _stage1(x, Wq, gq, keys[0], keys[1])
    table = memories.reshape(NUM_MEM * HEADS, DH_V)
    tvs, g = _stage3a(x, Wg, Wv, gg, gv)
    rows = _sc_gather_kernel()(table, idx.reshape(_NW, _NCH, _CH))
    out = _stage3b(rows.reshape(KH, N_TOK, DH_V), w, tvs, g, Wo, go)
    return out.reshape(b, n, DIM)
